# baseline (speedup 1.0000x reference)
import functools

import jax
import jax.numpy as jnp
from jax import lax
from jax.experimental import pallas as pl
from jax.experimental.pallas import tpu as pltpu
from jax.experimental.pallas import tpu_sc as plsc

_N = 10000
_E = 320000
_D = 128

_NC = 2
_NS = 16
_NW = _NC * _NS
_L = 16
_HALF = _N // _NC
_EPW = _E // _NW
_CHB = 128
_SEGR = 80
_SEG = _SEGR * _CHB
_NPC = 6400
_NTRASH = _NPC - _HALF
_ZPT = _NPC // _NS
_OPC = 5120
_RPT = _OPC // _NS
_ZR = 80


def _route_body(src_hbm, dst_hbm, zseg_hbm, tseg_hbm,
                rsrc_hbm, rdst_hbm, cnt_hbm,
                src_v, dst_v, osrc0, odst0, osrc1, odst1, cnt_v):
    c = lax.axis_index("c")
    s = lax.axis_index("s")
    w = s * _NC + c

    pltpu.sync_copy(src_hbm.at[w], src_v)
    pltpu.sync_copy(dst_hbm.at[w], dst_v)
    pltpu.sync_copy(zseg_hbm, osrc0)
    pltpu.sync_copy(tseg_hbm, odst0)
    pltpu.sync_copy(zseg_hbm, osrc1)
    pltpu.sync_copy(tseg_hbm, odst1)

    lane = jnp.arange(_L, dtype=jnp.int32)
    ones = jnp.ones((_L,), jnp.int32)
    zeros = jnp.zeros((_L,), jnp.int32)

    def body(i, offs):
        off0, off1 = offs
        sv = src_v[pl.ds(i * _L, _L)]
        dv = dst_v[pl.ds(i * _L, _L)]
        m0 = dv < _HALF
        m0i = jnp.where(m0, ones, zeros)
        cum0 = plsc.cumsum(m0i)
        n0 = jnp.sum(m0i)
        pos0 = off0 + cum0 - 1
        plsc.store_scatter(osrc0, [pos0 >> 7, pos0 & 127], sv, mask=m0)
        plsc.store_scatter(odst0, [pos0 >> 7, pos0 & 127], dv, mask=m0)
        m1 = jnp.logical_not(m0)
        pos1 = off1 + (lane - cum0)
        plsc.store_scatter(osrc1, [pos1 >> 7, pos1 & 127], sv, mask=m1)
        plsc.store_scatter(odst1, [pos1 >> 7, pos1 & 127], dv - _HALF,
                           mask=m1)
        return (off0 + n0, off1 + (_L - n0))

    zi = jnp.int32(0)
    off0, off1 = lax.fori_loop(0, _EPW // _L, body, (zi, zi))

    pltpu.sync_copy(osrc0, rsrc_hbm.at[0, w])
    pltpu.sync_copy(odst0, rdst_hbm.at[0, w])
    pltpu.sync_copy(osrc1, rsrc_hbm.at[1, w])
    pltpu.sync_copy(odst1, rdst_hbm.at[1, w])
    for k in range(_CHB // _L):
        cnt_v[pl.ds(k * _L, _L)] = jnp.full((_L,), 1, jnp.int32) * off0
    pltpu.sync_copy(cnt_v, cnt_hbm.at[0, w])
    for k in range(_CHB // _L):
        cnt_v[pl.ds(k * _L, _L)] = jnp.full((_L,), 1, jnp.int32) * off1
    pltpu.sync_copy(cnt_v, cnt_hbm.at[1, w])


_route = pl.kernel(
    _route_body,
    out_type=[
        jax.ShapeDtypeStruct((_NC, _NW, _SEGR, _CHB), jnp.int32),
        jax.ShapeDtypeStruct((_NC, _NW, _SEGR, _CHB), jnp.int32),
        jax.ShapeDtypeStruct((_NC, _NW, _CHB), jnp.int32),
    ],
    mesh=plsc.VectorSubcoreMesh(core_axis_name="c", subcore_axis_name="s"),
    compiler_params=pltpu.CompilerParams(needs_layout_passes=False),
    scratch_types=[
        pltpu.VMEM((_EPW,), jnp.int32),
        pltpu.VMEM((_EPW,), jnp.int32),
        pltpu.VMEM((_SEGR, _CHB), jnp.int32),
        pltpu.VMEM((_SEGR, _CHB), jnp.int32),
        pltpu.VMEM((_SEGR, _CHB), jnp.int32),
        pltpu.VMEM((_SEGR, _CHB), jnp.int32),
        pltpu.VMEM((_CHB,), jnp.int32),
    ],
)


def _agg_body(h_hbm, rsrc_hbm, rdst_hbm, cnt_hbm, zero_hbm, out_hbm,
              segs_v, segd_v, rows_v, cnt_s, acc_sh, gsem0, gsem1):
    c = lax.axis_index("c")
    s = lax.axis_index("s")

    for k in range(_ZPT // _ZR):
        pltpu.sync_copy(zero_hbm, acc_sh.at[pl.ds(s * _ZPT + k * _ZR, _ZR)])
    plsc.subcore_barrier()

    gsems = (gsem0, gsem1)

    def _start_gather(j, b):
        pltpu.async_copy(h_hbm.at[segs_v.at[j]], rows_v.at[b], gsems[b])

    def _wait_gather(j, b):
        pltpu.make_async_copy(h_hbm.at[segs_v.at[j]], rows_v.at[b],
                              gsems[b]).wait()

    for q in range(2):
        seg = 2 * s + q
        pltpu.sync_copy(cnt_hbm.at[c, seg], cnt_s)
        cvec = cnt_s[pl.ds(0, _L)]
        nch2 = _SEGR // 2
        pltpu.sync_copy(rsrc_hbm.at[c, seg], segs_v)
        pltpu.sync_copy(rdst_hbm.at[c, seg], segd_v)

        @pl.when(nch2 > 0)
        def _():
            _start_gather(0, 0)

            def body(i, carry):
                for b in range(2):
                    j = 2 * i + b

                    @pl.when(j + 1 < 2 * nch2)
                    def _():
                        _start_gather(j + 1, 1 - b)

                    _wait_gather(j, b)
                    pltpu.sync_copy(rows_v.at[b], acc_sh.at[segd_v.at[j]],
                                    add=True)
                return carry

            lax.fori_loop(0, nch2, body, 0)

    plsc.subcore_barrier()
    pltpu.sync_copy(acc_sh.at[pl.ds(s * _RPT, _RPT)],
                    out_hbm.at[c, pl.ds(s * _RPT, _RPT)])


_agg = pl.kernel(
    _agg_body,
    out_type=jax.ShapeDtypeStruct((_NC, _OPC, _D), jnp.float32),
    mesh=plsc.VectorSubcoreMesh(core_axis_name="c", subcore_axis_name="s"),
    scratch_types=[
        pltpu.VMEM((_SEGR, _CHB), jnp.int32),
        pltpu.VMEM((_SEGR, _CHB), jnp.int32),
        pltpu.VMEM((2, _CHB, _D), jnp.float32),
        pltpu.VMEM((_CHB,), jnp.int32),
        pltpu.VMEM_SHARED((_NPC, _D), jnp.float32),
        pltpu.SemaphoreType.DMA,
        pltpu.SemaphoreType.DMA,
    ],
)


_BLK = 1000
_GRID = _N // _BLK
_BPC = _HALF // _BLK

_row_spec = pl.BlockSpec((_BLK, _D), lambda i: (i, 0))
_p_spec = pl.BlockSpec((1, _BLK, _D), lambda i: (i // _BPC, i % _BPC, 0))
_w_spec = pl.BlockSpec((_D, _D), lambda i: (0, 0))
_b_spec = pl.BlockSpec((1, _D), lambda i: (0, 0))
_out_struct = jax.ShapeDtypeStruct((_N, _D), jnp.float32)


def _mm_body(x_ref, w_ref, o_ref):
    o_ref[...] = jnp.dot(x_ref[...], w_ref[...],
                         preferred_element_type=jnp.float32)


def _relu_mm_body(p_ref, b_ref, w_ref, o_ref):
    z = jnp.maximum(p_ref[0] + b_ref[...], 0.0)
    o_ref[...] = jnp.dot(z, w_ref[...], preferred_element_type=jnp.float32)


def _sigmoid_body(p_ref, b_ref, o_ref):
    o_ref[...] = jax.nn.sigmoid(p_ref[0] + b_ref[...])


_mm = pl.pallas_call(
    _mm_body, grid=(_GRID,),
    in_specs=[_row_spec, _w_spec],
    out_specs=_row_spec, out_shape=_out_struct)

_relu_mm = pl.pallas_call(
    _relu_mm_body, grid=(_GRID,),
    in_specs=[_p_spec, _b_spec, _w_spec],
    out_specs=_row_spec, out_shape=_out_struct)

_sigmoid = pl.pallas_call(
    _sigmoid_body, grid=(_GRID,),
    in_specs=[_p_spec, _b_spec],
    out_specs=_row_spec, out_shape=_out_struct)


def kernel(x, edge_index, W1, b1, W2, b2, W3, b3):
    src = edge_index[0].reshape(_NW, _EPW)
    dst = edge_index[1].reshape(_NW, _EPW)
    zseg = jnp.zeros((_SEGR, _CHB), jnp.int32)
    tseg = (_HALF + (jnp.arange(_SEG, dtype=jnp.int32) % _NTRASH)
            ).reshape(_SEGR, _CHB)
    zero = jnp.zeros((_ZR, _D), jnp.float32)

    rsrc, rdst, cnt = _route(src, dst, zseg, tseg)

    h = _mm(x, W1)
    p = _agg(h, rsrc, rdst, cnt, zero)
    h = _relu_mm(p, b1.reshape(1, _D), W2)
    p = _agg(h, rsrc, rdst, cnt, zero)
    h = _relu_mm(p, b2.reshape(1, _D), W3)
    p = _agg(h, rsrc, rdst, cnt, zero)
    return _sigmoid(p, b3.reshape(1, _D))


# speedup vs baseline: 88.2814x; 88.2814x over previous
import functools

import jax
import jax.numpy as jnp
from jax import lax
from jax.experimental import pallas as pl
from jax.experimental.pallas import tpu as pltpu
from jax.experimental.pallas import tpu_sc as plsc

_N = 10000
_E = 320000
_D = 128

_NC = 2
_NS = 16
_NW = _NC * _NS
_L = 16
_HALF = _N // _NC
_EPW = _E // _NW
_CHB = 128
_SEGR = 80
_SEG = _SEGR * _CHB
_NPC = 6400
_NTRASH = _NPC - _HALF
_ZPT = _NPC // _NS
_OPC = 5120
_RPT = _OPC // _NS
_ZR = 80


def _route_body(src_hbm, dst_hbm, zseg_hbm, tseg_hbm,
                rsrc_hbm, rdst_hbm, cnt_hbm,
                src_v, dst_v, osrc0, odst0, osrc1, odst1, cnt_v):
    c = lax.axis_index("c")
    s = lax.axis_index("s")
    w = s * _NC + c

    pltpu.sync_copy(src_hbm.at[w], src_v)
    pltpu.sync_copy(dst_hbm.at[w], dst_v)
    pltpu.sync_copy(zseg_hbm, osrc0)
    pltpu.sync_copy(tseg_hbm, odst0)
    pltpu.sync_copy(zseg_hbm, osrc1)
    pltpu.sync_copy(tseg_hbm, odst1)

    lane = jnp.arange(_L, dtype=jnp.int32)
    ones = jnp.ones((_L,), jnp.int32)
    zeros = jnp.zeros((_L,), jnp.int32)

    def body(i, offs):
        off0, off1 = offs
        sv = src_v[pl.ds(i * _L, _L)]
        dv = dst_v[pl.ds(i * _L, _L)]
        m0 = dv < _HALF
        m0i = jnp.where(m0, ones, zeros)
        cum0 = plsc.cumsum(m0i)
        n0 = jnp.sum(m0i)
        pos0 = off0 + cum0 - 1
        plsc.store_scatter(osrc0, [pos0 >> 7, pos0 & 127], sv, mask=m0)
        plsc.store_scatter(odst0, [pos0 >> 7, pos0 & 127], dv, mask=m0)
        m1 = jnp.logical_not(m0)
        pos1 = off1 + (lane - cum0)
        plsc.store_scatter(osrc1, [pos1 >> 7, pos1 & 127], sv, mask=m1)
        plsc.store_scatter(odst1, [pos1 >> 7, pos1 & 127], dv - _HALF,
                           mask=m1)
        return (off0 + n0, off1 + (_L - n0))

    zi = jnp.int32(0)
    off0, off1 = lax.fori_loop(0, _EPW // _L, body, (zi, zi))

    pltpu.sync_copy(osrc0, rsrc_hbm.at[0, w])
    pltpu.sync_copy(odst0, rdst_hbm.at[0, w])
    pltpu.sync_copy(osrc1, rsrc_hbm.at[1, w])
    pltpu.sync_copy(odst1, rdst_hbm.at[1, w])
    for k in range(_CHB // _L):
        cnt_v[pl.ds(k * _L, _L)] = jnp.full((_L,), 1, jnp.int32) * off0
    pltpu.sync_copy(cnt_v, cnt_hbm.at[0, w])
    for k in range(_CHB // _L):
        cnt_v[pl.ds(k * _L, _L)] = jnp.full((_L,), 1, jnp.int32) * off1
    pltpu.sync_copy(cnt_v, cnt_hbm.at[1, w])


_route = pl.kernel(
    _route_body,
    out_type=[
        jax.ShapeDtypeStruct((_NC, _NW, _SEGR, _CHB), jnp.int32),
        jax.ShapeDtypeStruct((_NC, _NW, _SEGR, _CHB), jnp.int32),
        jax.ShapeDtypeStruct((_NC, _NW, _CHB), jnp.int32),
    ],
    mesh=plsc.VectorSubcoreMesh(core_axis_name="c", subcore_axis_name="s"),
    compiler_params=pltpu.CompilerParams(needs_layout_passes=False),
    scratch_types=[
        pltpu.VMEM((_EPW,), jnp.int32),
        pltpu.VMEM((_EPW,), jnp.int32),
        pltpu.VMEM((_SEGR, _CHB), jnp.int32),
        pltpu.VMEM((_SEGR, _CHB), jnp.int32),
        pltpu.VMEM((_SEGR, _CHB), jnp.int32),
        pltpu.VMEM((_SEGR, _CHB), jnp.int32),
        pltpu.VMEM((_CHB,), jnp.int32),
    ],
)


def _agg_body(h_hbm, rsrc_hbm, rdst_hbm, cnt_hbm, zero_hbm, out_hbm,
              segs_v, segd_v, rows_v, cnt_s, acc_sh, gsem0, gsem1):
    c = lax.axis_index("c")
    s = lax.axis_index("s")

    for k in range(_ZPT // _ZR):
        pltpu.sync_copy(zero_hbm, acc_sh.at[pl.ds(s * _ZPT + k * _ZR, _ZR)])
    plsc.subcore_barrier()

    gsems = (gsem0, gsem1)

    def _start_gather(j, b):
        pltpu.async_copy(h_hbm.at[segs_v.at[j]], rows_v.at[b], gsems[b])

    def _wait_gather(j, b):
        pltpu.make_async_copy(h_hbm.at[segs_v.at[j]], rows_v.at[b],
                              gsems[b]).wait()

    for q in range(2):
        seg = 2 * s + q
        pltpu.sync_copy(cnt_hbm.at[c, seg], cnt_s)
        cvec = cnt_s[pl.ds(0, _L)]
        nch2 = (cvec[0] + 2 * _CHB - 1) // (2 * _CHB)
        pltpu.sync_copy(rsrc_hbm.at[c, seg], segs_v)
        pltpu.sync_copy(rdst_hbm.at[c, seg], segd_v)

        @pl.when(nch2 > 0)
        def _():
            _start_gather(0, 0)

            def body(i, carry):
                for b in range(2):
                    j = 2 * i + b

                    @pl.when(j + 1 < 2 * nch2)
                    def _():
                        _start_gather(j + 1, 1 - b)

                    _wait_gather(j, b)
                    pltpu.sync_copy(rows_v.at[b], acc_sh.at[segd_v.at[j]],
                                    add=True)
                return carry

            lax.fori_loop(0, nch2, body, 0)

    plsc.subcore_barrier()
    pltpu.sync_copy(acc_sh.at[pl.ds(s * _RPT, _RPT)],
                    out_hbm.at[c, pl.ds(s * _RPT, _RPT)])


_agg = pl.kernel(
    _agg_body,
    out_type=jax.ShapeDtypeStruct((_NC, _OPC, _D), jnp.float32),
    mesh=plsc.VectorSubcoreMesh(core_axis_name="c", subcore_axis_name="s"),
    scratch_types=[
        pltpu.VMEM((_SEGR, _CHB), jnp.int32),
        pltpu.VMEM((_SEGR, _CHB), jnp.int32),
        pltpu.VMEM((2, _CHB, _D), jnp.float32),
        pltpu.VMEM((_CHB,), jnp.int32),
        pltpu.VMEM_SHARED((_NPC, _D), jnp.float32),
        pltpu.SemaphoreType.DMA,
        pltpu.SemaphoreType.DMA,
    ],
)


_BLK = 1000
_GRID = _N // _BLK
_BPC = _HALF // _BLK

_row_spec = pl.BlockSpec((_BLK, _D), lambda i: (i, 0))
_p_spec = pl.BlockSpec((1, _BLK, _D), lambda i: (i // _BPC, i % _BPC, 0))
_w_spec = pl.BlockSpec((_D, _D), lambda i: (0, 0))
_b_spec = pl.BlockSpec((1, _D), lambda i: (0, 0))
_out_struct = jax.ShapeDtypeStruct((_N, _D), jnp.float32)


def _mm_body(x_ref, w_ref, o_ref):
    o_ref[...] = jnp.dot(x_ref[...], w_ref[...],
                         preferred_element_type=jnp.float32)


def _relu_mm_body(p_ref, b_ref, w_ref, o_ref):
    z = jnp.maximum(p_ref[0] + b_ref[...], 0.0)
    o_ref[...] = jnp.dot(z, w_ref[...], preferred_element_type=jnp.float32)


def _sigmoid_body(p_ref, b_ref, o_ref):
    o_ref[...] = jax.nn.sigmoid(p_ref[0] + b_ref[...])


_mm = pl.pallas_call(
    _mm_body, grid=(_GRID,),
    in_specs=[_row_spec, _w_spec],
    out_specs=_row_spec, out_shape=_out_struct)

_relu_mm = pl.pallas_call(
    _relu_mm_body, grid=(_GRID,),
    in_specs=[_p_spec, _b_spec, _w_spec],
    out_specs=_row_spec, out_shape=_out_struct)

_sigmoid = pl.pallas_call(
    _sigmoid_body, grid=(_GRID,),
    in_specs=[_p_spec, _b_spec],
    out_specs=_row_spec, out_shape=_out_struct)


def kernel(x, edge_index, W1, b1, W2, b2, W3, b3):
    src = edge_index[0].reshape(_NW, _EPW)
    dst = edge_index[1].reshape(_NW, _EPW)
    zseg = (jnp.arange(_SEG, dtype=jnp.int32) % _N).reshape(_SEGR, _CHB)
    tseg = (_HALF + (jnp.arange(_SEG, dtype=jnp.int32) % _NTRASH)
            ).reshape(_SEGR, _CHB)
    zero = jnp.zeros((_ZR, _D), jnp.float32)

    rsrc, rdst, cnt = _route(src, dst, zseg, tseg)

    h = _mm(x, W1)
    p = _agg(h, rsrc, rdst, cnt, zero)
    h = _relu_mm(p, b1.reshape(1, _D), W2)
    p = _agg(h, rsrc, rdst, cnt, zero)
    h = _relu_mm(p, b2.reshape(1, _D), W3)
    p = _agg(h, rsrc, rdst, cnt, zero)
    return _sigmoid(p, b3.reshape(1, _D))


# speedup vs baseline: 93.8316x; 1.0629x over previous
import functools

import jax
import jax.numpy as jnp
from jax import lax
from jax.experimental import pallas as pl
from jax.experimental.pallas import tpu as pltpu
from jax.experimental.pallas import tpu_sc as plsc

_N = 10000
_E = 320000
_D = 128

_NC = 2
_NS = 16
_NW = _NC * _NS
_L = 16
_HALF = _N // _NC
_EPW = _E // _NW
_CHB = 128
_SEGR = 80
_SEG = _SEGR * _CHB
_NPC = 6400
_NTRASH = _NPC - _HALF
_ZPT = _NPC // _NS
_OPC = 5120
_RPT = _OPC // _NS
_ZR = 80


def _route_body(src_hbm, dst_hbm, zseg_hbm, tseg_hbm,
                rsrc_hbm, rdst_hbm, cnt_hbm,
                src_v, dst_v, osrc0, odst0, osrc1, odst1, cnt_v):
    c = lax.axis_index("c")
    s = lax.axis_index("s")
    w = s * _NC + c

    pltpu.sync_copy(src_hbm.at[w], src_v)
    pltpu.sync_copy(dst_hbm.at[w], dst_v)
    pltpu.sync_copy(zseg_hbm, osrc0)
    pltpu.sync_copy(tseg_hbm, odst0)
    pltpu.sync_copy(zseg_hbm, osrc1)
    pltpu.sync_copy(tseg_hbm, odst1)

    lane = jnp.arange(_L, dtype=jnp.int32)
    ones = jnp.ones((_L,), jnp.int32)
    zeros = jnp.zeros((_L,), jnp.int32)

    def body(i, offs):
        off0, off1 = offs
        sv = src_v[pl.ds(i * _L, _L)]
        dv = dst_v[pl.ds(i * _L, _L)]
        m0 = dv < _HALF
        m0i = jnp.where(m0, ones, zeros)
        cum0 = plsc.cumsum(m0i)
        n0 = jnp.sum(m0i)
        pos0 = off0 + cum0 - 1
        plsc.store_scatter(osrc0, [pos0 >> 7, pos0 & 127], sv, mask=m0)
        plsc.store_scatter(odst0, [pos0 >> 7, pos0 & 127], dv, mask=m0)
        m1 = jnp.logical_not(m0)
        pos1 = off1 + (lane - cum0)
        plsc.store_scatter(osrc1, [pos1 >> 7, pos1 & 127], sv, mask=m1)
        plsc.store_scatter(odst1, [pos1 >> 7, pos1 & 127], dv - _HALF,
                           mask=m1)
        return (off0 + n0, off1 + (_L - n0))

    zi = jnp.int32(0)
    off0, off1 = lax.fori_loop(0, _EPW // _L, body, (zi, zi))

    pltpu.sync_copy(osrc0, rsrc_hbm.at[0, w])
    pltpu.sync_copy(odst0, rdst_hbm.at[0, w])
    pltpu.sync_copy(osrc1, rsrc_hbm.at[1, w])
    pltpu.sync_copy(odst1, rdst_hbm.at[1, w])
    for k in range(_CHB // _L):
        cnt_v[pl.ds(k * _L, _L)] = jnp.full((_L,), 1, jnp.int32) * off0
    pltpu.sync_copy(cnt_v, cnt_hbm.at[0, w])
    for k in range(_CHB // _L):
        cnt_v[pl.ds(k * _L, _L)] = jnp.full((_L,), 1, jnp.int32) * off1
    pltpu.sync_copy(cnt_v, cnt_hbm.at[1, w])


_route = pl.kernel(
    _route_body,
    out_type=[
        jax.ShapeDtypeStruct((_NC, _NW, _SEGR, _CHB), jnp.int32),
        jax.ShapeDtypeStruct((_NC, _NW, _SEGR, _CHB), jnp.int32),
        jax.ShapeDtypeStruct((_NC, _NW, _CHB), jnp.int32),
    ],
    mesh=plsc.VectorSubcoreMesh(core_axis_name="c", subcore_axis_name="s"),
    compiler_params=pltpu.CompilerParams(needs_layout_passes=False),
    scratch_types=[
        pltpu.VMEM((_EPW,), jnp.int32),
        pltpu.VMEM((_EPW,), jnp.int32),
        pltpu.VMEM((_SEGR, _CHB), jnp.int32),
        pltpu.VMEM((_SEGR, _CHB), jnp.int32),
        pltpu.VMEM((_SEGR, _CHB), jnp.int32),
        pltpu.VMEM((_SEGR, _CHB), jnp.int32),
        pltpu.VMEM((_CHB,), jnp.int32),
    ],
)


def _agg_body(h_hbm, rsrc_hbm, rdst_hbm, cnt_hbm, zero_hbm, out_hbm,
              segs_v, segd_v, rows_v, cnt_s, acc_sh, gsem0, gsem1, gsem2):
    c = lax.axis_index("c")
    s = lax.axis_index("s")

    for k in range(_ZPT // _ZR):
        pltpu.sync_copy(zero_hbm, acc_sh.at[pl.ds(s * _ZPT + k * _ZR, _ZR)])
    plsc.subcore_barrier()

    gsems = (gsem0, gsem1, gsem2)

    def _start_gather(j, b):
        pltpu.async_copy(h_hbm.at[segs_v.at[j]], rows_v.at[b], gsems[b])

    def _wait_gather(j, b):
        pltpu.make_async_copy(h_hbm.at[segs_v.at[j]], rows_v.at[b],
                              gsems[b]).wait()

    for q in range(2):
        seg = 2 * s + q
        pltpu.sync_copy(cnt_hbm.at[c, seg], cnt_s)
        cvec = cnt_s[pl.ds(0, _L)]
        nch3 = (cvec[0] + 3 * _CHB - 1) // (3 * _CHB)
        pltpu.sync_copy(rsrc_hbm.at[c, seg], segs_v)
        pltpu.sync_copy(rdst_hbm.at[c, seg], segd_v)

        @pl.when(nch3 > 0)
        def _():
            _start_gather(0, 0)
            _start_gather(1, 1)

            def body(i, carry):
                for b in range(3):
                    j = 3 * i + b

                    @pl.when(j + 2 < 3 * nch3)
                    def _():
                        _start_gather(j + 2, (b + 2) % 3)

                    _wait_gather(j, b)
                    pltpu.sync_copy(rows_v.at[b], acc_sh.at[segd_v.at[j]],
                                    add=True)
                return carry

            lax.fori_loop(0, nch3, body, 0)

    plsc.subcore_barrier()
    pltpu.sync_copy(acc_sh.at[pl.ds(s * _RPT, _RPT)],
                    out_hbm.at[c, pl.ds(s * _RPT, _RPT)])


_agg = pl.kernel(
    _agg_body,
    out_type=jax.ShapeDtypeStruct((_NC, _OPC, _D), jnp.float32),
    mesh=plsc.VectorSubcoreMesh(core_axis_name="c", subcore_axis_name="s"),
    scratch_types=[
        pltpu.VMEM((_SEGR, _CHB), jnp.int32),
        pltpu.VMEM((_SEGR, _CHB), jnp.int32),
        pltpu.VMEM((3, _CHB, _D), jnp.float32),
        pltpu.VMEM((_CHB,), jnp.int32),
        pltpu.VMEM_SHARED((_NPC, _D), jnp.float32),
        pltpu.SemaphoreType.DMA,
        pltpu.SemaphoreType.DMA,
        pltpu.SemaphoreType.DMA,
    ],
)


_BLK = 1000
_GRID = _N // _BLK
_BPC = _HALF // _BLK

_row_spec = pl.BlockSpec((_BLK, _D), lambda i: (i, 0))
_p_spec = pl.BlockSpec((1, _BLK, _D), lambda i: (i // _BPC, i % _BPC, 0))
_w_spec = pl.BlockSpec((_D, _D), lambda i: (0, 0))
_b_spec = pl.BlockSpec((1, _D), lambda i: (0, 0))
_out_struct = jax.ShapeDtypeStruct((_N, _D), jnp.float32)


def _mm_body(x_ref, w_ref, o_ref):
    o_ref[...] = jnp.dot(x_ref[...], w_ref[...],
                         preferred_element_type=jnp.float32)


def _relu_mm_body(p_ref, b_ref, w_ref, o_ref):
    z = jnp.maximum(p_ref[0] + b_ref[...], 0.0)
    o_ref[...] = jnp.dot(z, w_ref[...], preferred_element_type=jnp.float32)


def _sigmoid_body(p_ref, b_ref, o_ref):
    o_ref[...] = jax.nn.sigmoid(p_ref[0] + b_ref[...])


_mm = pl.pallas_call(
    _mm_body, grid=(_GRID,),
    in_specs=[_row_spec, _w_spec],
    out_specs=_row_spec, out_shape=_out_struct)

_relu_mm = pl.pallas_call(
    _relu_mm_body, grid=(_GRID,),
    in_specs=[_p_spec, _b_spec, _w_spec],
    out_specs=_row_spec, out_shape=_out_struct)

_sigmoid = pl.pallas_call(
    _sigmoid_body, grid=(_GRID,),
    in_specs=[_p_spec, _b_spec],
    out_specs=_row_spec, out_shape=_out_struct)


def kernel(x, edge_index, W1, b1, W2, b2, W3, b3):
    src = edge_index[0].reshape(_NW, _EPW)
    dst = edge_index[1].reshape(_NW, _EPW)
    zseg = (jnp.arange(_SEG, dtype=jnp.int32) % _N).reshape(_SEGR, _CHB)
    tseg = (_HALF + (jnp.arange(_SEG, dtype=jnp.int32) % _NTRASH)
            ).reshape(_SEGR, _CHB)
    zero = jnp.zeros((_ZR, _D), jnp.float32)

    rsrc, rdst, cnt = _route(src, dst, zseg, tseg)

    h = _mm(x, W1)
    p = _agg(h, rsrc, rdst, cnt, zero)
    h = _relu_mm(p, b1.reshape(1, _D), W2)
    p = _agg(h, rsrc, rdst, cnt, zero)
    h = _relu_mm(p, b2.reshape(1, _D), W3)
    p = _agg(h, rsrc, rdst, cnt, zero)
    return _sigmoid(p, b3.reshape(1, _D))


# speedup vs baseline: 94.6341x; 1.0086x over previous
import functools

import jax
import jax.numpy as jnp
from jax import lax
from jax.experimental import pallas as pl
from jax.experimental.pallas import tpu as pltpu
from jax.experimental.pallas import tpu_sc as plsc

_N = 10000
_E = 320000
_D = 128

_NC = 2
_NS = 16
_NW = _NC * _NS
_L = 16
_HALF = _N // _NC
_EPW = _E // _NW
_CHB = 128
_SEGR = 80
_SEG = _SEGR * _CHB
_NPC = 6400
_NTRASH = _NPC - _HALF
_ZPT = _NPC // _NS
_OPC = 5120
_RPT = _OPC // _NS
_ZR = 80


def _route_body(src_hbm, dst_hbm, zseg_hbm, tseg_hbm,
                rsrc_hbm, rdst_hbm, cnt_hbm,
                src_v, dst_v, osrc0, odst0, osrc1, odst1, cnt_v):
    c = lax.axis_index("c")
    s = lax.axis_index("s")
    w = s * _NC + c

    pltpu.sync_copy(src_hbm.at[w], src_v)
    pltpu.sync_copy(dst_hbm.at[w], dst_v)
    pltpu.sync_copy(zseg_hbm, osrc0)
    pltpu.sync_copy(tseg_hbm, odst0)
    pltpu.sync_copy(zseg_hbm, osrc1)
    pltpu.sync_copy(tseg_hbm, odst1)

    lane = jnp.arange(_L, dtype=jnp.int32)
    ones = jnp.ones((_L,), jnp.int32)
    zeros = jnp.zeros((_L,), jnp.int32)

    def body(i, offs):
        off0, off1 = offs
        sv = src_v[pl.ds(i * _L, _L)]
        dv = dst_v[pl.ds(i * _L, _L)]
        m0 = dv < _HALF
        m0i = jnp.where(m0, ones, zeros)
        cum0 = plsc.cumsum(m0i)
        n0 = jnp.sum(m0i)
        pos0 = off0 + cum0 - 1
        plsc.store_scatter(osrc0, [pos0 >> 7, pos0 & 127], sv, mask=m0)
        plsc.store_scatter(odst0, [pos0 >> 7, pos0 & 127], dv, mask=m0)
        m1 = jnp.logical_not(m0)
        pos1 = off1 + (lane - cum0)
        plsc.store_scatter(osrc1, [pos1 >> 7, pos1 & 127], sv, mask=m1)
        plsc.store_scatter(odst1, [pos1 >> 7, pos1 & 127], dv - _HALF,
                           mask=m1)
        return (off0 + n0, off1 + (_L - n0))

    zi = jnp.int32(0)
    off0, off1 = lax.fori_loop(0, _EPW // _L, body, (zi, zi))

    pltpu.sync_copy(osrc0, rsrc_hbm.at[0, w])
    pltpu.sync_copy(odst0, rdst_hbm.at[0, w])
    pltpu.sync_copy(osrc1, rsrc_hbm.at[1, w])
    pltpu.sync_copy(odst1, rdst_hbm.at[1, w])
    for k in range(_CHB // _L):
        cnt_v[pl.ds(k * _L, _L)] = jnp.full((_L,), 1, jnp.int32) * off0
    pltpu.sync_copy(cnt_v, cnt_hbm.at[0, w])
    for k in range(_CHB // _L):
        cnt_v[pl.ds(k * _L, _L)] = jnp.full((_L,), 1, jnp.int32) * off1
    pltpu.sync_copy(cnt_v, cnt_hbm.at[1, w])


_route = pl.kernel(
    _route_body,
    out_type=[
        jax.ShapeDtypeStruct((_NC, _NW, _SEGR, _CHB), jnp.int32),
        jax.ShapeDtypeStruct((_NC, _NW, _SEGR, _CHB), jnp.int32),
        jax.ShapeDtypeStruct((_NC, _NW, _CHB), jnp.int32),
    ],
    mesh=plsc.VectorSubcoreMesh(core_axis_name="c", subcore_axis_name="s"),
    compiler_params=pltpu.CompilerParams(needs_layout_passes=False),
    scratch_types=[
        pltpu.VMEM((_EPW,), jnp.int32),
        pltpu.VMEM((_EPW,), jnp.int32),
        pltpu.VMEM((_SEGR, _CHB), jnp.int32),
        pltpu.VMEM((_SEGR, _CHB), jnp.int32),
        pltpu.VMEM((_SEGR, _CHB), jnp.int32),
        pltpu.VMEM((_SEGR, _CHB), jnp.int32),
        pltpu.VMEM((_CHB,), jnp.int32),
    ],
)


def _agg_body(h_hbm, rsrc_hbm, rdst_hbm, cnt_hbm, zero_hbm, out_hbm,
              segs_v, segd_v, rows_v, cnt_s, acc_sh, gsem0, gsem1, gsem2):
    c = lax.axis_index("c")
    s = lax.axis_index("s")

    for k in range(_ZPT // _ZR):
        pltpu.sync_copy(zero_hbm, acc_sh.at[pl.ds(s * _ZPT + k * _ZR, _ZR)])
    plsc.subcore_barrier()

    gsems = (gsem0, gsem1, gsem2)

    def _start_gather(j, b):
        pltpu.async_copy(h_hbm.at[segs_v.at[j]], rows_v.at[b], gsems[b])

    def _wait_gather(j, b):
        pltpu.make_async_copy(h_hbm.at[segs_v.at[j]], rows_v.at[b],
                              gsems[b]).wait()

    for q in range(2):
        seg = 2 * s + q
        pltpu.sync_copy(cnt_hbm.at[c, seg], cnt_s)
        cvec = cnt_s[pl.ds(0, _L)]
        nch3 = (cvec[0] + 3 * _CHB - 1) // (3 * _CHB)
        pltpu.sync_copy(rsrc_hbm.at[c, seg], segs_v)
        pltpu.sync_copy(rdst_hbm.at[c, seg], segd_v)

        @pl.when(nch3 > 0)
        def _():
            _start_gather(0, 0)
            _start_gather(1, 1)

            def body(i, carry):
                for b in range(3):
                    j = 3 * i + b

                    @pl.when(j + 2 < 3 * nch3)
                    def _():
                        _start_gather(j + 2, (b + 2) % 3)

                    _wait_gather(j, b)
                    pltpu.sync_copy(rows_v.at[b], acc_sh.at[segd_v.at[j]],
                                    add=True)
                return carry

            lax.fori_loop(0, nch3, body, 0)

    plsc.subcore_barrier()
    pltpu.sync_copy(acc_sh.at[pl.ds(s * _RPT, _RPT)],
                    out_hbm.at[c, pl.ds(s * _RPT, _RPT)])


_agg = pl.kernel(
    _agg_body,
    out_type=jax.ShapeDtypeStruct((_NC, _OPC, _D), jnp.float32),
    mesh=plsc.VectorSubcoreMesh(core_axis_name="c", subcore_axis_name="s"),
    scratch_types=[
        pltpu.VMEM((_SEGR, _CHB), jnp.int32),
        pltpu.VMEM((_SEGR, _CHB), jnp.int32),
        pltpu.VMEM((3, _CHB, _D), jnp.float32),
        pltpu.VMEM((_CHB,), jnp.int32),
        pltpu.VMEM_SHARED((_NPC, _D), jnp.float32),
        pltpu.SemaphoreType.DMA,
        pltpu.SemaphoreType.DMA,
        pltpu.SemaphoreType.DMA,
    ],
)


_BLK = 1000
_GRID = _N // _BLK
_BPC = _HALF // _BLK

_row_spec = pl.BlockSpec((_BLK, _D), lambda i: (i, 0))
_p_spec = pl.BlockSpec((1, _BLK, _D), lambda i: (i // _BPC, i % _BPC, 0))
_w_spec = pl.BlockSpec((_D, _D), lambda i: (0, 0))
_b_spec = pl.BlockSpec((1, _D), lambda i: (0, 0))
_out_struct = jax.ShapeDtypeStruct((_N, _D), jnp.float32)


def _mm_relu_body(p_ref, w_ref, b_ref, o_ref):
    z = jnp.dot(p_ref[0], w_ref[...], preferred_element_type=jnp.float32)
    o_ref[...] = jnp.maximum(z + b_ref[...], 0.0)


def _mm_sigmoid_body(p_ref, w_ref, b_ref, o_ref):
    z = jnp.dot(p_ref[0], w_ref[...], preferred_element_type=jnp.float32)
    o_ref[...] = jax.nn.sigmoid(z + b_ref[...])


_mm_relu = pl.pallas_call(
    _mm_relu_body, grid=(_GRID,),
    in_specs=[_p_spec, _w_spec, _b_spec],
    out_specs=_row_spec, out_shape=_out_struct)

_mm_sigmoid = pl.pallas_call(
    _mm_sigmoid_body, grid=(_GRID,),
    in_specs=[_p_spec, _w_spec, _b_spec],
    out_specs=_row_spec, out_shape=_out_struct)


def kernel(x, edge_index, W1, b1, W2, b2, W3, b3):
    src = edge_index[0].reshape(_NW, _EPW)
    dst = edge_index[1].reshape(_NW, _EPW)
    zseg = (jnp.arange(_SEG, dtype=jnp.int32) % _N).reshape(_SEGR, _CHB)
    tseg = (_HALF + (jnp.arange(_SEG, dtype=jnp.int32) % _NTRASH)
            ).reshape(_SEGR, _CHB)
    zero = jnp.zeros((_ZR, _D), jnp.float32)

    rsrc, rdst, cnt = _route(src, dst, zseg, tseg)

    p = _agg(x, rsrc, rdst, cnt, zero)
    h = _mm_relu(p, W1, b1.reshape(1, _D))
    p = _agg(h, rsrc, rdst, cnt, zero)
    h = _mm_relu(p, W2, b2.reshape(1, _D))
    p = _agg(h, rsrc, rdst, cnt, zero)
    return _mm_sigmoid(p, W3, b3.reshape(1, _D))
